# baseline (speedup 1.0000x reference)
import functools

import jax
import jax.numpy as jnp
from jax import lax
from jax.experimental import pallas as pl
from jax.experimental.pallas import tpu as pltpu
from jax.experimental.pallas import tpu_sc as plsc

N = 10000
E = 320000
D = 128
NUM_CORR = 50
ALPHA_C = 0.5
NUM_SMOOTH = 50
ALPHA_S = 0.8
SCALE = 1.0

NT = 16
NPAD = 10240
SL = NPAD // NT
SR = SL // 16
NR = NPAD // 16
NRW = NPAD // 128
SRW = SL // 128
EPT = E // NT


def _matvec_body(x_ref, w_ref, o_ref):
    o_ref[...] = jnp.dot(x_ref[...], w_ref[...],
                         preferred_element_type=jnp.float32)


def _tc_matvec(xpad, W):
    return pl.pallas_call(
        _matvec_body,
        out_shape=jax.ShapeDtypeStruct((NPAD, 1), jnp.float32),
    )(xpad, W)


def _logit_body(p_ref, o_ref):
    p = jnp.clip(p_ref[...], 1e-7, 1.0 - 1e-7)
    o_ref[...] = jnp.log(p) - jnp.log1p(-p)


def _tc_logit(p2d):
    return pl.pallas_call(
        _logit_body,
        out_shape=jax.ShapeDtypeStruct((NPAD // 128, 128), jnp.float32),
    )(p2d)


_mesh = plsc.VectorSubcoreMesh(core_axis_name="c", subcore_axis_name="s",
                               num_cores=1)


@functools.partial(
    pl.kernel,
    out_type=jax.ShapeDtypeStruct((NPAD,), jnp.float32),
    mesh=_mesh,
    compiler_params=pltpu.CompilerParams(needs_layout_passes=False),
    scratch_types=[
        pltpu.VMEM((EPT,), jnp.int32),
        pltpu.VMEM((EPT,), jnp.int32),
        pltpu.VMEM((EPT,), jnp.int32),
        pltpu.VMEM((NPAD,), jnp.float32),
        pltpu.VMEM((NRW, 128), jnp.float32),
        pltpu.VMEM((SRW, 128), jnp.float32),
        pltpu.VMEM((SRW, 128), jnp.float32),
        pltpu.VMEM((1, NRW), jnp.int32),
        pltpu.VMEM((SL,), jnp.float32),
        pltpu.VMEM((SL,), jnp.float32),
        pltpu.VMEM((SL,), jnp.float32),
        pltpu.VMEM((SL,), jnp.float32),
        pltpu.VMEM((SL,), jnp.float32),
        pltpu.VMEM((SL,), jnp.float32),
        pltpu.VMEM((SL,), jnp.float32),
        pltpu.VMEM((16,), jnp.float32),
        pltpu.VMEM_SHARED((NRW, 128), jnp.float32),
        pltpu.VMEM_SHARED((NPAD,), jnp.float32),
        pltpu.VMEM_SHARED((NRW, 128), jnp.float32),
        pltpu.SemaphoreType.DMA,
    ],
)
def _sc_propagate(src_h, dst_h, xw_h, mask_h, ytrue_h, b_h, out_h,
                  e_src, e_dst, e_pk, zf, acc, sl2, zb2, idx2d, mask_s,
                  ytrue_s, err_s, probs_s, out_s, dinv_s, zout_s, bb,
                  sh_acc, sh_y, sh_zero, zsem):
    wid = lax.axis_index("s")
    base = wid * SL
    base_r = wid * SRW

    def edge_accum():
        @plsc.parallel_loop(0, EPT, 16, unroll=32)
        def _body(i):
            pk = e_pk[pl.ds(i, 16)]
            s = lax.bitwise_and(pk, 0x3FFF)
            hi = lax.shift_right_logical(pk, 21)
            lo = lax.bitwise_and(lax.shift_right_logical(pk, 14), 127)
            plsc.addupdate_scatter(acc, [hi, lo], plsc.load_gather(zf, [s]))

    def reduce_slices(post):
        pltpu.sync_copy(acc, sh_acc.at[idx2d.at[0]], add=True)
        zc = pltpu.async_copy(sh_zero, acc, zsem)
        plsc.subcore_barrier()
        pltpu.sync_copy(sh_acc.at[pl.ds(base_r, SRW)], sl2)
        pltpu.sync_copy(zb2, sh_acc.at[pl.ds(base_r, SRW)])
        for j in range(SRW):
            for k in range(8):
                post(pl.ds((j * 8 + k) * 16, 16), sl2[j, pl.ds(k * 16, 16)])
        return zc

    def publish_z(src_ref):
        pltpu.sync_copy(src_ref, sh_y.at[pl.ds(base, SL)])
        plsc.subcore_barrier()
        pltpu.sync_copy(sh_y, zf)

    pltpu.sync_copy(src_h.at[pl.ds(wid * EPT, EPT)], e_src)
    pltpu.sync_copy(dst_h.at[pl.ds(wid * EPT, EPT)], e_dst)
    pltpu.sync_copy(mask_h.at[pl.ds(base, SL)], mask_s)
    pltpu.sync_copy(ytrue_h.at[pl.ds(base, SL)], ytrue_s)
    pltpu.sync_copy(b_h, bb)

    for r in range(SRW):
        for k in range(8):
            zb2[r, pl.ds(k * 16, 16)] = jnp.zeros((16,), jnp.float32)
    for k in range(NRW // 16):
        idx2d[0, pl.ds(k * 16, 16)] = (
            k * 16 + lax.broadcasted_iota(jnp.int32, (16,), 0))
    pltpu.sync_copy(zb2, sh_acc.at[pl.ds(base_r, SRW)])
    pltpu.sync_copy(zb2, sh_zero.at[pl.ds(base_r, SRW)])
    plsc.subcore_barrier()
    pltpu.sync_copy(sh_zero, acc)

    @plsc.parallel_loop(0, EPT, 16, unroll=8)
    def _pack(i):
        sl = pl.ds(i, 16)
        e_pk[sl] = lax.bitwise_or(lax.shift_left(e_dst[sl], 14), e_src[sl])

    @plsc.parallel_loop(0, EPT, 16, unroll=8)
    def _deg(i):
        d = e_dst[pl.ds(i, 16)]
        hi = lax.shift_right_logical(d, 7)
        lo = lax.bitwise_and(d, 127)
        plsc.addupdate_scatter(acc, [hi, lo], jnp.ones((16,), jnp.float32))

    def dinv_post(sl, v):
        dg = jnp.maximum(v, 1.0)
        xi = plsc.bitcast(dg, jnp.int32)
        xi = 0x5F3759DF - lax.shift_right_arithmetic(xi, 1)
        r = plsc.bitcast(xi, jnp.float32)
        r = r * (1.5 - 0.5 * dg * r * r)
        r = r * (1.5 - 0.5 * dg * r * r)
        r = r * (1.5 - 0.5 * dg * r * r)
        dinv_s[sl] = r
    zc = reduce_slices(dinv_post)

    pltpu.sync_copy(xw_h.at[pl.ds(base, SL)], out_s)

    def zxw_f(j, _):
        sl = pl.ds(j * 16, 16)
        zout_s[sl] = dinv_s[sl] * out_s[sl]
        return 0
    lax.fori_loop(0, SR, zxw_f, 0)
    publish_z(zout_s)
    zc.wait()
    edge_accum()

    def head_post(sl, v):
        lg = dinv_s[sl] * v + bb[pl.ds(0, 16)]
        p = 1.0 / (1.0 + jnp.exp(-lg))
        probs_s[sl] = p
        e = mask_s[sl] * (ytrue_s[sl] - p)
        err_s[sl] = e
        zout_s[sl] = dinv_s[sl] * e
    zc = reduce_slices(head_post)

    publish_z(zout_s)
    zc.wait()

    def corr_post(sl, v):
        o = ALPHA_C * (dinv_s[sl] * v) + (1.0 - ALPHA_C) * err_s[sl]
        m = mask_s[sl]
        o = m * err_s[sl] + (1.0 - m) * o
        out_s[sl] = o
        zout_s[sl] = dinv_s[sl] * o

    def corr_body(k, _):
        edge_accum()
        zck = reduce_slices(corr_post)
        publish_z(zout_s)
        zck.wait()
        return 0
    lax.fori_loop(0, NUM_CORR, corr_body, 0)

    def sminit(j, _):
        sl = pl.ds(j * 16, 16)
        c = probs_s[sl] + SCALE * out_s[sl]
        m = mask_s[sl]
        ysm = m * ytrue_s[sl] + (1.0 - m) * c
        err_s[sl] = (1.0 - ALPHA_S) * ysm
        out_s[sl] = ysm
        zout_s[sl] = dinv_s[sl] * ysm
        return 0
    lax.fori_loop(0, SR, sminit, 0)
    publish_z(zout_s)

    def sm_post(sl, v):
        o = ALPHA_S * (dinv_s[sl] * v) + err_s[sl]
        o = jnp.minimum(jnp.maximum(o, 0.0), 1.0)
        out_s[sl] = o
        zout_s[sl] = dinv_s[sl] * o

    def sm_body(k, _):
        with jax.named_scope("sm_accum"):
            edge_accum()
        with jax.named_scope("sm_reduce"):
            zck = reduce_slices(sm_post)
        with jax.named_scope("sm_publish"):
            publish_z(zout_s)
            zck.wait()
        return 0
    lax.fori_loop(0, NUM_SMOOTH, sm_body, 0)

    pltpu.sync_copy(out_s, out_h.at[pl.ds(base, SL)])


def kernel(x, edge_index, W, b, train_mask, train_labels):
    src = edge_index[0]
    dst = edge_index[1]
    xpad = jnp.pad(x, ((0, NPAD - N), (0, 0)))
    xw = _tc_matvec(xpad, W.astype(jnp.float32)).reshape(NPAD)
    maskf = jnp.pad(train_mask.astype(jnp.float32), (0, NPAD - N))
    ytrue = jnp.pad(train_labels.astype(jnp.float32), (0, NPAD - N))
    bvec = jnp.broadcast_to(b.astype(jnp.float32), (16,))
    probs2 = _sc_propagate(src, dst, xw, maskf, ytrue, bvec)
    logits = _tc_logit(probs2.reshape(NPAD // 128, 128))
    return logits.reshape(NPAD)[:N].reshape(N, 1)


# speedup vs baseline: 1.0021x; 1.0021x over previous
import functools

import jax
import jax.numpy as jnp
from jax import lax
from jax.experimental import pallas as pl
from jax.experimental.pallas import tpu as pltpu
from jax.experimental.pallas import tpu_sc as plsc

N = 10000
E = 320000
D = 128
NUM_CORR = 50
ALPHA_C = 0.5
NUM_SMOOTH = 50
ALPHA_S = 0.8
SCALE = 1.0

NT = 16
NPAD = 10240
SL = NPAD // NT
SR = SL // 16
NR = NPAD // 16
NRW = NPAD // 128
SRW = SL // 128
EPT = E // NT


def _matvec_body(x_ref, w_ref, o_ref):
    o_ref[...] = jnp.dot(x_ref[...], w_ref[...],
                         preferred_element_type=jnp.float32)


def _tc_matvec(xpad, W):
    return pl.pallas_call(
        _matvec_body,
        out_shape=jax.ShapeDtypeStruct((NPAD, 1), jnp.float32),
    )(xpad, W)


def _logit_body(p_ref, o_ref):
    p = jnp.clip(p_ref[...], 1e-7, 1.0 - 1e-7)
    o_ref[...] = jnp.log(p) - jnp.log1p(-p)


def _tc_logit(p2d):
    return pl.pallas_call(
        _logit_body,
        out_shape=jax.ShapeDtypeStruct((NPAD // 128, 128), jnp.float32),
    )(p2d)


_mesh = plsc.VectorSubcoreMesh(core_axis_name="c", subcore_axis_name="s",
                               num_cores=1)


@functools.partial(
    pl.kernel,
    out_type=jax.ShapeDtypeStruct((NPAD,), jnp.float32),
    mesh=_mesh,
    compiler_params=pltpu.CompilerParams(needs_layout_passes=False),
    scratch_types=[
        pltpu.VMEM((EPT,), jnp.int32),
        pltpu.VMEM((EPT,), jnp.int32),
        pltpu.VMEM((EPT,), jnp.int32),
        pltpu.VMEM((NPAD,), jnp.float32),
        pltpu.VMEM((NRW, 128), jnp.float32),
        pltpu.VMEM((SRW, 128), jnp.float32),
        pltpu.VMEM((SRW, 128), jnp.float32),
        pltpu.VMEM((1, NRW), jnp.int32),
        pltpu.VMEM((SL,), jnp.float32),
        pltpu.VMEM((SL,), jnp.float32),
        pltpu.VMEM((SL,), jnp.float32),
        pltpu.VMEM((SL,), jnp.float32),
        pltpu.VMEM((SL,), jnp.float32),
        pltpu.VMEM((SL,), jnp.float32),
        pltpu.VMEM((SL,), jnp.float32),
        pltpu.VMEM((16,), jnp.float32),
        pltpu.VMEM_SHARED((NRW, 128), jnp.float32),
        pltpu.VMEM_SHARED((NPAD,), jnp.float32),
        pltpu.VMEM_SHARED((NRW, 128), jnp.float32),
        pltpu.SemaphoreType.DMA,
    ],
)
def _sc_propagate(src_h, dst_h, xw_h, mask_h, ytrue_h, b_h, out_h,
                  e_src, e_dst, e_pk, zf, acc, sl2, zb2, idx2d, mask_s,
                  ytrue_s, err_s, probs_s, out_s, dinv_s, zout_s, bb,
                  sh_acc, sh_y, sh_zero, zsem):
    wid = lax.axis_index("s")
    base = wid * SL
    base_r = wid * SRW

    def edge_accum():
        @plsc.parallel_loop(0, EPT, 16, unroll=16)
        def _body(i):
            pk = e_pk[pl.ds(i, 16)]
            s = lax.bitwise_and(pk, 0x3FFF)
            hi = lax.shift_right_logical(pk, 21)
            lo = lax.bitwise_and(lax.shift_right_logical(pk, 14), 127)
            plsc.addupdate_scatter(acc, [hi, lo], plsc.load_gather(zf, [s]))

    def reduce_slices(post):
        pltpu.sync_copy(acc, sh_acc.at[idx2d.at[0]], add=True)
        zc = pltpu.async_copy(sh_zero, acc, zsem)
        plsc.subcore_barrier()
        pltpu.sync_copy(sh_acc.at[pl.ds(base_r, SRW)], sl2)
        pltpu.sync_copy(zb2, sh_acc.at[pl.ds(base_r, SRW)])
        for j in range(SRW):
            for k in range(8):
                post(pl.ds((j * 8 + k) * 16, 16), sl2[j, pl.ds(k * 16, 16)])
        return zc

    def publish_z(src_ref):
        pltpu.sync_copy(src_ref, sh_y.at[pl.ds(base, SL)])
        plsc.subcore_barrier()
        pltpu.sync_copy(sh_y, zf)

    pltpu.sync_copy(src_h.at[pl.ds(wid * EPT, EPT)], e_src)
    pltpu.sync_copy(dst_h.at[pl.ds(wid * EPT, EPT)], e_dst)
    pltpu.sync_copy(mask_h.at[pl.ds(base, SL)], mask_s)
    pltpu.sync_copy(ytrue_h.at[pl.ds(base, SL)], ytrue_s)
    pltpu.sync_copy(b_h, bb)

    for r in range(SRW):
        for k in range(8):
            zb2[r, pl.ds(k * 16, 16)] = jnp.zeros((16,), jnp.float32)
    for k in range(NRW // 16):
        idx2d[0, pl.ds(k * 16, 16)] = (
            k * 16 + lax.broadcasted_iota(jnp.int32, (16,), 0))
    pltpu.sync_copy(zb2, sh_acc.at[pl.ds(base_r, SRW)])
    pltpu.sync_copy(zb2, sh_zero.at[pl.ds(base_r, SRW)])
    plsc.subcore_barrier()
    pltpu.sync_copy(sh_zero, acc)

    @plsc.parallel_loop(0, EPT, 16, unroll=8)
    def _pack(i):
        sl = pl.ds(i, 16)
        e_pk[sl] = lax.bitwise_or(lax.shift_left(e_dst[sl], 14), e_src[sl])

    @plsc.parallel_loop(0, EPT, 16, unroll=8)
    def _deg(i):
        d = e_dst[pl.ds(i, 16)]
        hi = lax.shift_right_logical(d, 7)
        lo = lax.bitwise_and(d, 127)
        plsc.addupdate_scatter(acc, [hi, lo], jnp.ones((16,), jnp.float32))

    def dinv_post(sl, v):
        dg = jnp.maximum(v, 1.0)
        xi = plsc.bitcast(dg, jnp.int32)
        xi = 0x5F3759DF - lax.shift_right_arithmetic(xi, 1)
        r = plsc.bitcast(xi, jnp.float32)
        r = r * (1.5 - 0.5 * dg * r * r)
        r = r * (1.5 - 0.5 * dg * r * r)
        r = r * (1.5 - 0.5 * dg * r * r)
        dinv_s[sl] = r
    zc = reduce_slices(dinv_post)

    pltpu.sync_copy(xw_h.at[pl.ds(base, SL)], out_s)

    def zxw_f(j, _):
        sl = pl.ds(j * 16, 16)
        zout_s[sl] = dinv_s[sl] * out_s[sl]
        return 0
    lax.fori_loop(0, SR, zxw_f, 0)
    publish_z(zout_s)
    zc.wait()
    edge_accum()

    def head_post(sl, v):
        lg = dinv_s[sl] * v + bb[pl.ds(0, 16)]
        p = 1.0 / (1.0 + jnp.exp(-lg))
        probs_s[sl] = p
        e = mask_s[sl] * (ytrue_s[sl] - p)
        err_s[sl] = e
        zout_s[sl] = dinv_s[sl] * e
    zc = reduce_slices(head_post)

    publish_z(zout_s)
    zc.wait()

    def corr_post(sl, v):
        o = ALPHA_C * (dinv_s[sl] * v) + (1.0 - ALPHA_C) * err_s[sl]
        m = mask_s[sl]
        o = m * err_s[sl] + (1.0 - m) * o
        out_s[sl] = o
        zout_s[sl] = dinv_s[sl] * o

    def corr_body(k, _):
        edge_accum()
        zck = reduce_slices(corr_post)
        publish_z(zout_s)
        zck.wait()
        return 0
    lax.fori_loop(0, NUM_CORR, corr_body, 0)

    def sminit(j, _):
        sl = pl.ds(j * 16, 16)
        c = probs_s[sl] + SCALE * out_s[sl]
        m = mask_s[sl]
        ysm = m * ytrue_s[sl] + (1.0 - m) * c
        err_s[sl] = (1.0 - ALPHA_S) * ysm
        out_s[sl] = ysm
        zout_s[sl] = dinv_s[sl] * ysm
        return 0
    lax.fori_loop(0, SR, sminit, 0)
    publish_z(zout_s)

    def sm_post(sl, v):
        o = ALPHA_S * (dinv_s[sl] * v) + err_s[sl]
        o = jnp.minimum(jnp.maximum(o, 0.0), 1.0)
        out_s[sl] = o
        zout_s[sl] = dinv_s[sl] * o

    def sm_body(k, _):
        edge_accum()
        zck = reduce_slices(sm_post)
        publish_z(zout_s)
        zck.wait()
        return 0
    lax.fori_loop(0, NUM_SMOOTH, sm_body, 0)

    pltpu.sync_copy(out_s, out_h.at[pl.ds(base, SL)])


def kernel(x, edge_index, W, b, train_mask, train_labels):
    src = edge_index[0]
    dst = edge_index[1]
    xpad = jnp.pad(x, ((0, NPAD - N), (0, 0)))
    xw = _tc_matvec(xpad, W.astype(jnp.float32)).reshape(NPAD)
    maskf = jnp.pad(train_mask.astype(jnp.float32), (0, NPAD - N))
    ytrue = jnp.pad(train_labels.astype(jnp.float32), (0, NPAD - N))
    bvec = jnp.broadcast_to(b.astype(jnp.float32), (16,))
    probs2 = _sc_propagate(src, dst, xw, maskf, ytrue, bvec)
    logits = _tc_logit(probs2.reshape(NPAD // 128, 128))
    return logits.reshape(NPAD)[:N].reshape(N, 1)
